# baseline (speedup 1.0000x reference)
import functools

import jax
import jax.numpy as jnp
from jax import lax
from jax.experimental import pallas as pl
from jax.experimental.pallas import tpu as pltpu
from jax.experimental.pallas import tpu_sc as plsc


@functools.lru_cache(maxsize=None)
def _make_gather(V, D, B):
    info = plsc.get_sparse_core_info()
    NC, NS = info.num_cores, info.num_subcores
    NW = NC * NS
    assert B % NW == 0
    b_per_w = B // NW
    C = 800
    assert b_per_w % C == 0
    n_chunks = b_per_w // C
    mesh = plsc.VectorSubcoreMesh(core_axis_name="c", subcore_axis_name="s")

    @functools.partial(
        pl.kernel,
        mesh=mesh,
        out_type=jax.ShapeDtypeStruct((B, D), jnp.float32),
        scratch_types=[
            pltpu.VMEM((C,), jnp.int32),
            pltpu.VMEM((C, D), jnp.float32),
            pltpu.SemaphoreType.DMA,
        ],
        compiler_params=pltpu.CompilerParams(use_tc_tiling_on_sc=False),
    )
    def k(table_hbm, idx_hbm, out_hbm, idx_v, rows_v, sem):
        wid = lax.axis_index("s") * NC + lax.axis_index("c")
        base = wid * b_per_w
        for i in range(n_chunks):
            off = base + i * C
            pltpu.sync_copy(idx_hbm.at[pl.ds(off, C)], idx_v)
            pltpu.async_copy(table_hbm.at[idx_v], rows_v, sem).wait()
            pltpu.sync_copy(rows_v, out_hbm.at[pl.ds(off, C)])

    return k


def kernel(input_, weight):
    S0, S1 = input_.shape
    B = S0 * S1
    V, D = weight.shape
    idx = input_.reshape(B).astype(jnp.int32)
    out = _make_gather(V, D, B)(weight, idx)
    return out.reshape(S0, S1, D)


# speedup vs baseline: 1.0123x; 1.0123x over previous
import functools

import jax
import jax.numpy as jnp
from jax import lax
from jax.experimental import pallas as pl
from jax.experimental.pallas import tpu as pltpu
from jax.experimental.pallas import tpu_sc as plsc


@functools.lru_cache(maxsize=None)
def _make_gather(V, D, B):
    info = plsc.get_sparse_core_info()
    NC, NS = info.num_cores, info.num_subcores
    NW = NC * NS
    assert B % NW == 0
    b_per_w = B // NW
    C = 400
    NBUF = 4
    assert b_per_w % C == 0
    n_chunks = b_per_w // C
    assert n_chunks % NBUF == 0 and n_chunks >= 2 * NBUF
    mesh = plsc.VectorSubcoreMesh(core_axis_name="c", subcore_axis_name="s")

    @functools.partial(
        pl.kernel,
        mesh=mesh,
        out_type=jax.ShapeDtypeStruct((B, D), jnp.float32),
        scratch_types=[
            pltpu.VMEM((b_per_w,), jnp.int32),
            [pltpu.VMEM((C, D), jnp.float32) for _ in range(NBUF)],
            [pltpu.SemaphoreType.DMA for _ in range(NBUF)],
            [pltpu.SemaphoreType.DMA for _ in range(NBUF)],
        ],
        compiler_params=pltpu.CompilerParams(use_tc_tiling_on_sc=False),
    )
    def k(table_hbm, idx_hbm, out_hbm, idx_v, rows, sem_g, sem_s):
        wid = lax.axis_index("s") * NC + lax.axis_index("c")
        base = wid * b_per_w
        pltpu.sync_copy(idx_hbm.at[pl.ds(base, b_per_w)], idx_v)

        def start_gather(chunk, b):
            idx_slice = idx_v.at[pl.ds(chunk * C, C)]
            return pltpu.async_copy(table_hbm.at[idx_slice], rows[b], sem_g[b])

        def start_store(chunk, b):
            return pltpu.async_copy(
                rows[b], out_hbm.at[pl.ds(base + chunk * C, C)], sem_s[b]
            )

        gathers = [start_gather(b, b) for b in range(NBUF)]
        stores = [None] * NBUF
        for i in range(n_chunks):
            b = i % NBUF
            gathers[b].wait()
            stores[b] = start_store(i, b)
            nxt = i + NBUF
            if nxt < n_chunks:
                stores[b].wait()
                gathers[b] = start_gather(nxt, b)
        for b in range(NBUF):
            stores[b].wait()

    return k


def kernel(input_, weight):
    S0, S1 = input_.shape
    B = S0 * S1
    V, D = weight.shape
    idx = input_.reshape(B).astype(jnp.int32)
    out = _make_gather(V, D, B)(weight, idx)
    return out.reshape(S0, S1, D)


# speedup vs baseline: 1.3091x; 1.2933x over previous
import functools

import jax
import jax.numpy as jnp
from jax import lax
from jax.experimental import pallas as pl
from jax.experimental.pallas import tpu as pltpu
from jax.experimental.pallas import tpu_sc as plsc


@functools.lru_cache(maxsize=None)
def _make_gather(V, D, B):
    info = plsc.get_sparse_core_info()
    NC, NS, L = info.num_cores, info.num_subcores, info.num_lanes
    NW = NC * NS
    assert B % NW == 0
    b_per_w = B // NW
    C = 160
    NBUF = 4
    assert b_per_w % C == 0 and C % L == 0
    n_chunks = b_per_w // C
    assert n_chunks >= 2 * NBUF
    mesh = plsc.VectorSubcoreMesh(core_axis_name="c", subcore_axis_name="s")

    @functools.partial(
        pl.kernel,
        mesh=mesh,
        out_type=jax.ShapeDtypeStruct((B, D), jnp.float32),
        scratch_types=[
            pltpu.VMEM((b_per_w,), jnp.int32),
            [pltpu.VMEM((C, D), jnp.float32) for _ in range(NBUF)],
            [pltpu.SemaphoreType.DMA for _ in range(NBUF)],
            [pltpu.SemaphoreType.DMA for _ in range(NBUF)],
        ],
        compiler_params=pltpu.CompilerParams(use_tc_tiling_on_sc=True),
    )
    def k(table_hbm, idx_hbm, out_hbm, idx_v, rows, sem_g, sem_s):
        wid = lax.axis_index("s") * NC + lax.axis_index("c")
        base = wid * b_per_w
        pltpu.sync_copy(idx_hbm.at[pl.ds(base, b_per_w)], idx_v)

        def start_gather(chunk, b):
            def grp(g, _):
                i16 = idx_v[pl.ds(chunk * C + g * L, L)]
                for l in range(L):
                    pltpu.async_copy(
                        table_hbm.at[pl.ds(i16[l], 1)],
                        rows[b].at[pl.ds(g * L + l, 1)],
                        sem_g[b],
                    )
                return 0

            lax.fori_loop(0, C // L, grp, 0)

        def drain_gather(b):
            def w(p, _):
                pltpu.make_async_copy(
                    table_hbm.at[pl.ds(0, 1)], rows[b].at[pl.ds(0, 1)], sem_g[b]
                ).wait()
                return 0

            lax.fori_loop(0, C, w, 0)

        def start_store(chunk, b):
            return pltpu.async_copy(
                rows[b], out_hbm.at[pl.ds(base + chunk * C, C)], sem_s[b]
            )

        for b in range(NBUF):
            start_gather(b, b)
        stores = [None] * NBUF
        for i in range(n_chunks):
            b = i % NBUF
            drain_gather(b)
            stores[b] = start_store(i, b)
            nxt = i + NBUF
            if nxt < n_chunks:
                stores[b].wait()
                start_gather(nxt, b)
        for b in range(NBUF):
            stores[b].wait()

    return k


def kernel(input_, weight):
    S0, S1 = input_.shape
    B = S0 * S1
    V, D = weight.shape
    idx = input_.reshape(B).astype(jnp.int32)
    out = _make_gather(V, D, B)(weight, idx)
    return out.reshape(S0, S1, D)
